# baseline (speedup 1.0000x reference)
import functools

import jax
import jax.numpy as jnp
from jax import lax
from jax.experimental import pallas as pl
from jax.experimental.pallas import tpu as pltpu
from jax.experimental.pallas import tpu_sc as plsc

NUM_EMB = 1_000_000
DIM = 32
BATCH = 16384
NUM_CORES = 2
NUM_SUBCORES = 16
NUM_TILES = NUM_CORES * NUM_SUBCORES
B_PER_TILE = BATCH // NUM_TILES
CHUNK = 128
NCHUNK = B_PER_TILE // CHUNK
LANES = 16


def _body(t0_hbm, t1_hbm, idx_hbm, hp_hbm, wb_hbm, out_hbm,
          idx_v, h_v, rows0_v, rows1_v, hp_v, wb_v, sem):
    wid = lax.axis_index("s") * NUM_CORES + lax.axis_index("c")
    base = wid * B_PER_TILE

    pltpu.sync_copy(idx_hbm.at[pl.ds(base, B_PER_TILE)], idx_v)
    pltpu.sync_copy(hp_hbm, hp_v)
    pltpu.sync_copy(wb_hbm, wb_v)

    a0 = hp_v[0, :]
    c0 = hp_v[1, :]
    b0 = hp_v[2, :]
    a1 = hp_v[3, :]
    c1 = hp_v[4, :]
    b1 = hp_v[5, :]

    inv_m = jnp.float32(1.0 / NUM_EMB)
    m = jnp.int32(NUM_EMB)

    def _mod_m(s):
        q = (s.astype(jnp.float32) * inv_m).astype(jnp.int32)
        r = s - q * m
        r = jnp.where(r < 0, r + m, r)
        r = jnp.where(r >= m, r - m, r)
        return r

    def hash_iter(j, carry):
        off = pl.multiple_of(j * LANES, LANES)
        x = idx_v[pl.ds(off, LANES)]
        r = _mod_m(x)
        x1 = r >> 10
        x0 = r & 1023
        chunk = j >> 3
        lane_off = pl.multiple_of((j & 7) * LANES, LANES)
        h_v[0, chunk, pl.ds(lane_off, LANES)] = _mod_m(x0 * a0 + x1 * c0 + b0)
        h_v[1, chunk, pl.ds(lane_off, LANES)] = _mod_m(x0 * a1 + x1 * c1 + b1)
        return carry

    lax.fori_loop(jnp.int32(0), jnp.int32(B_PER_TILE // LANES), hash_iter, 0)

    copies = []
    for j in range(NCHUNK):
        row = pl.ds(jnp.int32(j * CHUNK), CHUNK)
        copies.append(pltpu.async_copy(
            t0_hbm.at[h_v.at[jnp.int32(0), jnp.int32(j)]],
            rows0_v.at[row], sem))
        copies.append(pltpu.async_copy(
            t1_hbm.at[h_v.at[jnp.int32(1), jnp.int32(j)]],
            rows1_v.at[row], sem))
    for c in copies:
        c.wait()

    w0 = wb_v[0, :]
    w1 = wb_v[1, :]

    def comb_iter(k, carry):
        for half in range(DIM // LANES):
            sl = pl.ds(half * LANES, LANES)
            rows0_v[k, sl] = rows0_v[k, sl] * w0 + rows1_v[k, sl] * w1
        return carry

    lax.fori_loop(jnp.int32(0), jnp.int32(B_PER_TILE), comb_iter, 0)

    pltpu.sync_copy(rows0_v, out_hbm.at[pl.ds(base, B_PER_TILE)])


def kernel(indices, tables, weights, hash_a, hash_b):
    idx32 = indices.astype(jnp.int32)
    t0 = tables[0]
    t1 = tables[1]
    a_mod = jnp.mod(hash_a, NUM_EMB).astype(jnp.int32)
    b_mod = jnp.mod(hash_b, NUM_EMB).astype(jnp.int32)
    c_mod = jnp.mod(a_mod * 1024, NUM_EMB).astype(jnp.int32)
    hp = jnp.stack([a_mod[0], c_mod[0], b_mod[0],
                    a_mod[1], c_mod[1], b_mod[1]]).astype(jnp.int32)
    hp = jnp.broadcast_to(hp[:, None], (6, LANES))
    wb = jnp.broadcast_to(weights.astype(jnp.float32)[:, None], (2, LANES))

    mesh = plsc.VectorSubcoreMesh(
        core_axis_name="c", subcore_axis_name="s")
    run = pl.kernel(
        _body,
        out_type=jax.ShapeDtypeStruct((BATCH, DIM), jnp.float32),
        mesh=mesh,
        scratch_types=[
            pltpu.VMEM((B_PER_TILE,), jnp.int32),
            pltpu.VMEM((2, NCHUNK, CHUNK), jnp.int32),
            pltpu.VMEM((B_PER_TILE, DIM), jnp.float32),
            pltpu.VMEM((B_PER_TILE, DIM), jnp.float32),
            pltpu.VMEM((6, LANES), jnp.int32),
            pltpu.VMEM((2, LANES), jnp.float32),
            pltpu.SemaphoreType.DMA,
        ],
        compiler_params=pltpu.CompilerParams(use_tc_tiling_on_sc=False),
    )
    return run(t0, t1, idx32, hp, wb)


# speedup vs baseline: 1.0012x; 1.0012x over previous
import functools

import jax
import jax.numpy as jnp
from jax import lax
from jax.experimental import pallas as pl
from jax.experimental.pallas import tpu as pltpu
from jax.experimental.pallas import tpu_sc as plsc

NUM_EMB = 1_000_000
DIM = 32
BATCH = 16384
NUM_CORES = 2
NUM_SUBCORES = 16
NUM_TILES = NUM_CORES * NUM_SUBCORES
B_PER_TILE = BATCH // NUM_TILES
CHUNK = 128
NCHUNK = B_PER_TILE // CHUNK
LANES = 16


def _body(t0_hbm, t1_hbm, idx_hbm, hp_hbm, wb_hbm, out_hbm,
          idx_v, h_v, rows0_v, rows1_v, hp_v, wb_v, sem):
    wid = lax.axis_index("s") * NUM_CORES + lax.axis_index("c")
    base = wid * B_PER_TILE

    with jax.named_scope("copy_in"):
        pltpu.sync_copy(idx_hbm.at[pl.ds(base, B_PER_TILE)], idx_v)
        pltpu.sync_copy(hp_hbm, hp_v)
        pltpu.sync_copy(wb_hbm, wb_v)

    a0 = hp_v[0, :]
    c0 = hp_v[1, :]
    b0 = hp_v[2, :]
    a1 = hp_v[3, :]
    c1 = hp_v[4, :]
    b1 = hp_v[5, :]

    inv_m = jnp.float32(1.0 / NUM_EMB)
    m = jnp.int32(NUM_EMB)

    def _mod_m(s):
        q = (s.astype(jnp.float32) * inv_m).astype(jnp.int32)
        r = s - q * m
        r = jnp.where(r < 0, r + m, r)
        r = jnp.where(r >= m, r - m, r)
        return r

    def hash_iter(j, carry):
        off = pl.multiple_of(j * LANES, LANES)
        x = idx_v[pl.ds(off, LANES)]
        r = _mod_m(x)
        x1 = r >> 10
        x0 = r & 1023
        chunk = j >> 3
        lane_off = pl.multiple_of((j & 7) * LANES, LANES)
        h_v[0, chunk, pl.ds(lane_off, LANES)] = _mod_m(x0 * a0 + x1 * c0 + b0)
        h_v[1, chunk, pl.ds(lane_off, LANES)] = _mod_m(x0 * a1 + x1 * c1 + b1)
        return carry

    with jax.named_scope("hash"):
        lax.fori_loop(jnp.int32(0), jnp.int32(B_PER_TILE // LANES),
                      hash_iter, 0)

    with jax.named_scope("gather"):
        copies = []
        for j in range(NCHUNK):
            row = pl.ds(jnp.int32(j * CHUNK), CHUNK)
            copies.append(pltpu.async_copy(
                t0_hbm.at[h_v.at[jnp.int32(0), jnp.int32(j)]],
                rows0_v.at[row], sem))
            copies.append(pltpu.async_copy(
                t1_hbm.at[h_v.at[jnp.int32(1), jnp.int32(j)]],
                rows1_v.at[row], sem))
        for c in copies:
            c.wait()

    w0 = wb_v[0, :]
    w1 = wb_v[1, :]

    def comb_iter(k, carry):
        for half in range(DIM // LANES):
            sl = pl.ds(half * LANES, LANES)
            rows0_v[k, sl] = rows0_v[k, sl] * w0 + rows1_v[k, sl] * w1
        return carry

    with jax.named_scope("combine"):
        lax.fori_loop(jnp.int32(0), jnp.int32(B_PER_TILE), comb_iter, 0)

    with jax.named_scope("copy_out"):
        pltpu.sync_copy(rows0_v, out_hbm.at[pl.ds(base, B_PER_TILE)])


def kernel(indices, tables, weights, hash_a, hash_b):
    idx32 = indices.astype(jnp.int32)
    t0 = tables[0]
    t1 = tables[1]
    a_mod = jnp.mod(hash_a, NUM_EMB).astype(jnp.int32)
    b_mod = jnp.mod(hash_b, NUM_EMB).astype(jnp.int32)
    c_mod = jnp.mod(a_mod * 1024, NUM_EMB).astype(jnp.int32)
    hp = jnp.stack([a_mod[0], c_mod[0], b_mod[0],
                    a_mod[1], c_mod[1], b_mod[1]]).astype(jnp.int32)
    hp = jnp.broadcast_to(hp[:, None], (6, LANES))
    wb = jnp.broadcast_to(weights.astype(jnp.float32)[:, None], (2, LANES))

    mesh = plsc.VectorSubcoreMesh(
        core_axis_name="c", subcore_axis_name="s")
    run = pl.kernel(
        _body,
        out_type=jax.ShapeDtypeStruct((BATCH, DIM), jnp.float32),
        mesh=mesh,
        scratch_types=[
            pltpu.VMEM((B_PER_TILE,), jnp.int32),
            pltpu.VMEM((2, NCHUNK, CHUNK), jnp.int32),
            pltpu.VMEM((B_PER_TILE, DIM), jnp.float32),
            pltpu.VMEM((B_PER_TILE, DIM), jnp.float32),
            pltpu.VMEM((6, LANES), jnp.int32),
            pltpu.VMEM((2, LANES), jnp.float32),
            pltpu.SemaphoreType.DMA,
        ],
        compiler_params=pltpu.CompilerParams(use_tc_tiling_on_sc=False),
    )
    return run(t0, t1, idx32, hp, wb)


# speedup vs baseline: 1.4274x; 1.4257x over previous
import functools

import jax
import jax.numpy as jnp
from jax import lax
from jax.experimental import pallas as pl
from jax.experimental.pallas import tpu as pltpu
from jax.experimental.pallas import tpu_sc as plsc

NUM_EMB = 1_000_000
DIM = 32
BATCH = 16384
NUM_CORES = 2
NUM_SUBCORES = 16
NUM_TILES = NUM_CORES * NUM_SUBCORES
B_PER_TILE = BATCH // NUM_TILES
CHUNK = 128
NCHUNK = B_PER_TILE // CHUNK
LANES = 16


def _body(tables_hbm, idx_hbm, hp_hbm, wb_hbm, out_hbm,
          idx_v, h_v, rows0_v, rows1_v, hp_v, wb_v, sem):
    t0_hbm = tables_hbm.at[jnp.int32(0)]
    t1_hbm = tables_hbm.at[jnp.int32(1)]
    wid = lax.axis_index("s") * NUM_CORES + lax.axis_index("c")
    base = wid * B_PER_TILE

    with jax.named_scope("copy_in"):
        pltpu.sync_copy(idx_hbm.at[pl.ds(base, B_PER_TILE)], idx_v)
        pltpu.sync_copy(hp_hbm, hp_v)
        pltpu.sync_copy(wb_hbm, wb_v)

    a0 = hp_v[0, :]
    c0 = hp_v[1, :]
    b0 = hp_v[2, :]
    a1 = hp_v[3, :]
    c1 = hp_v[4, :]
    b1 = hp_v[5, :]

    inv_m = jnp.float32(1.0 / NUM_EMB)
    m = jnp.int32(NUM_EMB)

    def _mod_m(s):
        q = (s.astype(jnp.float32) * inv_m).astype(jnp.int32)
        r = s - q * m
        r = jnp.where(r < 0, r + m, r)
        r = jnp.where(r >= m, r - m, r)
        return r

    def hash_iter(j, carry):
        off = pl.multiple_of(j * LANES, LANES)
        x = idx_v[pl.ds(off, LANES)]
        r = _mod_m(x)
        x1 = r >> 10
        x0 = r & 1023
        chunk = j >> 3
        lane_off = pl.multiple_of((j & 7) * LANES, LANES)
        h_v[0, chunk, pl.ds(lane_off, LANES)] = _mod_m(x0 * a0 + x1 * c0 + b0)
        h_v[1, chunk, pl.ds(lane_off, LANES)] = _mod_m(x0 * a1 + x1 * c1 + b1)
        return carry

    with jax.named_scope("hash"):
        lax.fori_loop(jnp.int32(0), jnp.int32(B_PER_TILE // LANES),
                      hash_iter, 0)

    with jax.named_scope("gather"):
        copies = []
        for j in range(NCHUNK):
            row = pl.ds(jnp.int32(j * CHUNK), CHUNK)
            copies.append(pltpu.async_copy(
                t0_hbm.at[h_v.at[jnp.int32(0), jnp.int32(j)]],
                rows0_v.at[row], sem))
            copies.append(pltpu.async_copy(
                t1_hbm.at[h_v.at[jnp.int32(1), jnp.int32(j)]],
                rows1_v.at[row], sem))
        for c in copies:
            c.wait()

    w0 = wb_v[0, :]
    w1 = wb_v[1, :]

    def comb_iter(k, carry):
        for half in range(DIM // LANES):
            sl = pl.ds(half * LANES, LANES)
            rows0_v[k, sl] = rows0_v[k, sl] * w0 + rows1_v[k, sl] * w1
        return carry

    with jax.named_scope("combine"):
        lax.fori_loop(jnp.int32(0), jnp.int32(B_PER_TILE), comb_iter, 0)

    with jax.named_scope("copy_out"):
        pltpu.sync_copy(rows0_v, out_hbm.at[pl.ds(base, B_PER_TILE)])


def kernel(indices, tables, weights, hash_a, hash_b):
    idx32 = indices.astype(jnp.int32)
    a_mod = jnp.mod(hash_a, NUM_EMB).astype(jnp.int32)
    b_mod = jnp.mod(hash_b, NUM_EMB).astype(jnp.int32)
    c_mod = jnp.mod(a_mod * 1024, NUM_EMB).astype(jnp.int32)
    hp = jnp.stack([a_mod[0], c_mod[0], b_mod[0],
                    a_mod[1], c_mod[1], b_mod[1]]).astype(jnp.int32)
    hp = jnp.broadcast_to(hp[:, None], (6, LANES))
    wb = jnp.broadcast_to(weights.astype(jnp.float32)[:, None], (2, LANES))

    mesh = plsc.VectorSubcoreMesh(
        core_axis_name="c", subcore_axis_name="s")
    run = pl.kernel(
        _body,
        out_type=jax.ShapeDtypeStruct((BATCH, DIM), jnp.float32),
        mesh=mesh,
        scratch_types=[
            pltpu.VMEM((B_PER_TILE,), jnp.int32),
            pltpu.VMEM((2, NCHUNK, CHUNK), jnp.int32),
            pltpu.VMEM((B_PER_TILE, DIM), jnp.float32),
            pltpu.VMEM((B_PER_TILE, DIM), jnp.float32),
            pltpu.VMEM((6, LANES), jnp.int32),
            pltpu.VMEM((2, LANES), jnp.float32),
            pltpu.SemaphoreType.DMA,
        ],
        compiler_params=pltpu.CompilerParams(use_tc_tiling_on_sc=False),
    )
    return run(tables, idx32, hp, wb)
